# baseline (speedup 1.0000x reference)
import functools

import jax
import jax.numpy as jnp
from jax import lax
from jax.experimental import pallas as pl
from jax.experimental.pallas import tpu as pltpu
from jax.experimental.pallas import tpu_sc as plsc

NUM_CLASS = 100000
BATCH = 1024
CTX_DIM = 512
N_CLS_CTX = 4
PREFIX_LEN = 5
SUFFIX_LEN = 68
CLIP_LEN = 77
MID_START = PREFIX_LEN
SUF_START = PREFIX_LEN + N_CLS_CTX

NC, NS = 2, 16
NW = NC * NS
BPW = BATCH // NW
CHUNK = 16
LANES = 16
NKC = CTX_DIM // LANES

_mesh = plsc.VectorSubcoreMesh(core_axis_name="c", subcore_axis_name="s")


@functools.partial(
    pl.kernel,
    mesh=_mesh,
    out_type=jax.ShapeDtypeStruct((N_CLS_CTX, BATCH, CTX_DIM), jnp.float32),
    scratch_types=[
        pltpu.VMEM((BPW,), jnp.int32),
        pltpu.VMEM((CHUNK, N_CLS_CTX, CTX_DIM), jnp.float32),
        pltpu.VMEM((N_CLS_CTX, CHUNK, CTX_DIM), jnp.float32),
        pltpu.VMEM((N_CLS_CTX, CHUNK, CTX_DIM), jnp.float32),
        pltpu.SemaphoreType.DMA,
        pltpu.SemaphoreType.DMA,
    ],
    compiler_params=pltpu.CompilerParams(use_tc_tiling_on_sc=True),
)
def _sc_gather_t(label_hbm, table_hbm, out_hbm, idx_v, rows_v,
                 mid_a, mid_b, gsem, msem):
    wid = lax.axis_index("s") * NC + lax.axis_index("c")
    base = wid * BPW
    pltpu.sync_copy(label_hbm.at[pl.ds(base, BPW)], idx_v)
    mids = (mid_a, mid_b)
    for c in range(BPW // CHUNK):
        pltpu.async_copy(
            table_hbm.at[idx_v.at[pl.ds(c * CHUNK, CHUNK)]], rows_v,
            gsem).wait()
        midp = mids[c]

        def transpose_row(j, _, _midp=midp):
            for m in range(N_CLS_CTX):
                for k in range(NKC):
                    sl = pl.ds(k * LANES, LANES)
                    _midp[m, j, sl] = rows_v[j, m, sl]
            return ()
        lax.fori_loop(0, CHUNK, transpose_row, ())

        for m in range(N_CLS_CTX):
            pltpu.async_copy(
                midp.at[m], out_hbm.at[m, pl.ds(base + c * CHUNK, CHUNK)],
                msem)
    for c in range(BPW // CHUNK):
        for m in range(N_CLS_CTX):
            pltpu.make_async_copy(
                mids[c].at[m], out_hbm.at[m, pl.ds(base, CHUNK)], msem).wait()


N_BCAST = CLIP_LEN - N_CLS_CTX


BCB = 512


def _tc_broadcast_body(tok_ref, o_ref):
    o_ref[0] = jnp.broadcast_to(tok_ref[0], (BCB, CTX_DIM))


def _tc_broadcast(token_prefix, token_suffix):
    tokvec = jnp.concatenate(
        [token_prefix[0], token_suffix[0]], axis=0
    ).reshape(N_BCAST, 1, CTX_DIM)
    return pl.pallas_call(
        _tc_broadcast_body,
        grid=(N_BCAST, BATCH // BCB),
        in_specs=[pl.BlockSpec((1, 1, CTX_DIM), lambda t, j: (t, 0, 0))],
        out_specs=pl.BlockSpec(
            (1, BCB, CTX_DIM),
            lambda t, j: (jnp.where(t < MID_START, t, t + N_CLS_CTX), j, 0)),
        out_shape=jax.ShapeDtypeStruct((CLIP_LEN, BATCH, CTX_DIM), jnp.float32),
    )(tokvec)


def _tc_splice_body(g_ref, bc_ref, o_ref):
    del bc_ref
    o_ref[0] = g_ref[0]


def _tc_splice(gathered_t, bc):
    return pl.pallas_call(
        _tc_splice_body,
        grid=(N_CLS_CTX,),
        in_specs=[
            pl.BlockSpec((1, BATCH, CTX_DIM), lambda m: (m, 0, 0)),
            pl.BlockSpec(memory_space=pl.ANY),
        ],
        out_specs=pl.BlockSpec((1, BATCH, CTX_DIM),
                               lambda m: (m + MID_START, 0, 0)),
        out_shape=jax.ShapeDtypeStruct((CLIP_LEN, BATCH, CTX_DIM), jnp.float32),
        input_output_aliases={1: 0},
    )(gathered_t, bc)


def kernel(label, cls_ctx, token_prefix, token_suffix):
    gathered_t = _sc_gather_t(label.astype(jnp.int32), cls_ctx)
    bc = _tc_broadcast(token_prefix, token_suffix)
    out = _tc_splice(gathered_t, bc)
    return out.transpose(1, 0, 2)


# speedup vs baseline: 1.2341x; 1.2341x over previous
import functools

import jax
import jax.numpy as jnp
from jax import lax
from jax.experimental import pallas as pl
from jax.experimental.pallas import tpu as pltpu
from jax.experimental.pallas import tpu_sc as plsc

NUM_CLASS = 100000
BATCH = 1024
CTX_DIM = 512
N_CLS_CTX = 4
PREFIX_LEN = 5
SUFFIX_LEN = 68
CLIP_LEN = 77
MID_START = PREFIX_LEN
SUF_START = PREFIX_LEN + N_CLS_CTX

NC, NS = 2, 16
NW = NC * NS
BPW = BATCH // NW
CHUNK = 16
LANES = 16
NKC = CTX_DIM // LANES

_mesh = plsc.VectorSubcoreMesh(core_axis_name="c", subcore_axis_name="s")


@functools.partial(
    pl.kernel,
    mesh=_mesh,
    out_type=jax.ShapeDtypeStruct((N_CLS_CTX, BATCH, CTX_DIM), jnp.float32),
    scratch_types=[
        pltpu.VMEM((BPW,), jnp.int32),
        pltpu.VMEM((CHUNK, N_CLS_CTX, CTX_DIM), jnp.float32),
        pltpu.VMEM((N_CLS_CTX, CHUNK, CTX_DIM), jnp.float32),
        pltpu.VMEM((N_CLS_CTX, CHUNK, CTX_DIM), jnp.float32),
        pltpu.SemaphoreType.DMA,
        pltpu.SemaphoreType.DMA,
    ],
    compiler_params=pltpu.CompilerParams(use_tc_tiling_on_sc=True),
)
def _sc_gather_t(label_hbm, table_hbm, out_hbm, idx_v, rows_v,
                 mid_a, mid_b, gsem, msem):
    wid = lax.axis_index("s") * NC + lax.axis_index("c")
    base = wid * BPW
    pltpu.sync_copy(label_hbm.at[pl.ds(base, BPW)], idx_v)
    mids = (mid_a, mid_b)
    for c in range(BPW // CHUNK):
        pltpu.async_copy(
            table_hbm.at[idx_v.at[pl.ds(c * CHUNK, CHUNK)]], rows_v,
            gsem).wait()
        midp = mids[c]

        def transpose_row(j, _, _midp=midp):
            for m in range(N_CLS_CTX):
                for k in range(NKC):
                    sl = pl.ds(k * LANES, LANES)
                    _midp[m, j, sl] = rows_v[j, m, sl]
            return ()
        lax.fori_loop(0, CHUNK, transpose_row, ())

        for m in range(N_CLS_CTX):
            pltpu.async_copy(
                midp.at[m], out_hbm.at[m, pl.ds(base + c * CHUNK, CHUNK)],
                msem)
    for c in range(BPW // CHUNK):
        for m in range(N_CLS_CTX):
            pltpu.make_async_copy(
                mids[c].at[m], out_hbm.at[m, pl.ds(base, CHUNK)], msem).wait()


N_BCAST = CLIP_LEN - N_CLS_CTX


BCB = 1024


def _tc_broadcast_body(tok_ref, o_ref):
    o_ref[0] = jnp.broadcast_to(tok_ref[0], (BCB, CTX_DIM))


def _tc_broadcast(token_prefix, token_suffix):
    tokvec = jnp.concatenate(
        [token_prefix[0], token_suffix[0]], axis=0
    ).reshape(N_BCAST, 1, CTX_DIM)
    return pl.pallas_call(
        _tc_broadcast_body,
        grid=(N_BCAST, BATCH // BCB),
        in_specs=[pl.BlockSpec((1, 1, CTX_DIM), lambda t, j: (t, 0, 0))],
        out_specs=pl.BlockSpec(
            (1, BCB, CTX_DIM),
            lambda t, j: (jnp.where(t < MID_START, t, t + N_CLS_CTX), j, 0)),
        out_shape=jax.ShapeDtypeStruct((CLIP_LEN, BATCH, CTX_DIM), jnp.float32),
    )(tokvec)


def _tc_splice_body(g_ref, bc_ref, o_ref):
    del bc_ref
    o_ref[0] = g_ref[0]


def _tc_splice(gathered_t, bc):
    return pl.pallas_call(
        _tc_splice_body,
        grid=(N_CLS_CTX,),
        in_specs=[
            pl.BlockSpec((1, BATCH, CTX_DIM), lambda m: (m, 0, 0)),
            pl.BlockSpec(memory_space=pl.ANY),
        ],
        out_specs=pl.BlockSpec((1, BATCH, CTX_DIM),
                               lambda m: (m + MID_START, 0, 0)),
        out_shape=jax.ShapeDtypeStruct((CLIP_LEN, BATCH, CTX_DIM), jnp.float32),
        input_output_aliases={1: 0},
    )(gathered_t, bc)


def kernel(label, cls_ctx, token_prefix, token_suffix):
    gathered_t = _sc_gather_t(label.astype(jnp.int32), cls_ctx)
    bc = _tc_broadcast(token_prefix, token_suffix)
    out = _tc_splice(gathered_t, bc)
    return out.transpose(1, 0, 2)


# speedup vs baseline: 1.3939x; 1.1295x over previous
import functools

import jax
import jax.numpy as jnp
from jax import lax
from jax.experimental import pallas as pl
from jax.experimental.pallas import tpu as pltpu
from jax.experimental.pallas import tpu_sc as plsc

NUM_CLASS = 100000
BATCH = 1024
CTX_DIM = 512
N_CLS_CTX = 4
PREFIX_LEN = 5
SUFFIX_LEN = 68
CLIP_LEN = 77
MID_START = PREFIX_LEN
SUF_START = PREFIX_LEN + N_CLS_CTX

NC, NS = 2, 16
NW = NC * NS
BPW = BATCH // NW
CHUNK = 16
LANES = 16
NKC = CTX_DIM // LANES

_mesh = plsc.VectorSubcoreMesh(core_axis_name="c", subcore_axis_name="s")


@functools.partial(
    pl.kernel,
    mesh=_mesh,
    out_type=jax.ShapeDtypeStruct((N_CLS_CTX, BATCH, CTX_DIM), jnp.float32),
    scratch_types=[
        pltpu.VMEM((BPW,), jnp.int32),
        pltpu.VMEM((CHUNK, N_CLS_CTX, CTX_DIM), jnp.float32),
        pltpu.VMEM((N_CLS_CTX, CHUNK, CTX_DIM), jnp.float32),
        pltpu.VMEM((N_CLS_CTX, CHUNK, CTX_DIM), jnp.float32),
        pltpu.SemaphoreType.DMA,
        pltpu.SemaphoreType.DMA,
    ],
    compiler_params=pltpu.CompilerParams(use_tc_tiling_on_sc=True),
)
def _sc_gather_t(label_hbm, table_hbm, out_hbm, idx_v, rows_v,
                 mid_a, mid_b, gsem, msem):
    wid = lax.axis_index("s") * NC + lax.axis_index("c")
    base = wid * BPW
    pltpu.sync_copy(label_hbm.at[pl.ds(base, BPW)], idx_v)
    mids = (mid_a, mid_b)
    for c in range(BPW // CHUNK):
        pltpu.async_copy(
            table_hbm.at[idx_v.at[pl.ds(c * CHUNK, CHUNK)]], rows_v,
            gsem).wait()
        midp = mids[c]

        def transpose_row(j, _, _midp=midp):
            for m in range(N_CLS_CTX):
                for k in range(NKC):
                    sl = pl.ds(k * LANES, LANES)
                    _midp[m, j, sl] = rows_v[j, m, sl]
            return ()
        lax.fori_loop(0, CHUNK, transpose_row, ())

        for m in range(N_CLS_CTX):
            pltpu.async_copy(
                midp.at[m], out_hbm.at[m, pl.ds(base + c * CHUNK, CHUNK)],
                msem)
    for c in range(BPW // CHUNK):
        for m in range(N_CLS_CTX):
            pltpu.make_async_copy(
                mids[c].at[m], out_hbm.at[m, pl.ds(base, CHUNK)], msem).wait()


N_BCAST = CLIP_LEN - N_CLS_CTX


TPB = 7


def _tc_broadcast_body(tok_ref, o_ref):
    o_ref[...] = jnp.broadcast_to(tok_ref[...], (TPB, BATCH, CTX_DIM))


def _tc_broadcast(token_prefix, token_suffix):
    tokvec = jnp.concatenate(
        [token_prefix[0],
         jnp.broadcast_to(token_prefix[0, :N_CLS_CTX], (N_CLS_CTX, CTX_DIM)),
         token_suffix[0]], axis=0).reshape(CLIP_LEN, 1, CTX_DIM)
    return pl.pallas_call(
        _tc_broadcast_body,
        grid=(CLIP_LEN // TPB,),
        in_specs=[pl.BlockSpec((TPB, 1, CTX_DIM), lambda t: (t, 0, 0))],
        out_specs=pl.BlockSpec((TPB, BATCH, CTX_DIM), lambda t: (t, 0, 0)),
        out_shape=jax.ShapeDtypeStruct((CLIP_LEN, BATCH, CTX_DIM), jnp.float32),
    )(tokvec)


def _tc_splice_body(g_ref, bc_ref, o_ref):
    del bc_ref
    o_ref[0] = g_ref[0]


def _tc_splice(gathered_t, bc):
    return pl.pallas_call(
        _tc_splice_body,
        grid=(N_CLS_CTX,),
        in_specs=[
            pl.BlockSpec((1, BATCH, CTX_DIM), lambda m: (m, 0, 0)),
            pl.BlockSpec(memory_space=pl.ANY),
        ],
        out_specs=pl.BlockSpec((1, BATCH, CTX_DIM),
                               lambda m: (m + MID_START, 0, 0)),
        out_shape=jax.ShapeDtypeStruct((CLIP_LEN, BATCH, CTX_DIM), jnp.float32),
        input_output_aliases={1: 0},
    )(gathered_t, bc)


def kernel(label, cls_ctx, token_prefix, token_suffix):
    gathered_t = _sc_gather_t(label.astype(jnp.int32), cls_ctx)
    bc = _tc_broadcast(token_prefix, token_suffix)
    out = _tc_splice(gathered_t, bc)
    return out.transpose(1, 0, 2)
